# speedup vs baseline: 3.2501x; 3.2501x over previous
import jax
import jax.numpy as jnp
from jax.experimental import pallas as pl

SIZE = 4096
DIM = 512
B = 4
N = 4096
D_EMB = 512

BN = 512


def _copy_concat_kernel(emb_ref, pe_ref, out_ref):
    out_ref[:, :, :D_EMB] = emb_ref[...]
    out_ref[:, :, D_EMB:] = jnp.broadcast_to(pe_ref[...][None], (B, BN, DIM))


def kernel(emb, pe):
    pe_rows = jax.lax.slice(pe, (1, 0), (1 + N, DIM))
    grid = (N // BN,)
    return pl.pallas_call(
        _copy_concat_kernel,
        grid=grid,
        in_specs=[
            pl.BlockSpec((B, BN, D_EMB), lambda j: (0, j, 0)),
            pl.BlockSpec((BN, DIM), lambda j: (j, 0)),
        ],
        out_specs=pl.BlockSpec((B, BN, D_EMB + DIM), lambda j: (0, j, 0)),
        out_shape=jax.ShapeDtypeStruct((B, N, D_EMB + DIM), emb.dtype),
    )(emb, pe_rows)


# speedup vs baseline: 3.3892x; 1.0428x over previous
import jax
import jax.numpy as jnp
from jax.experimental import pallas as pl

SIZE = 4096
DIM = 512
B = 4
N = 4096
D_EMB = 512

BN = 1024


def _copy_concat_kernel(emb_ref, pe_ref, out_ref):
    out_ref[:, :, :D_EMB] = emb_ref[...]
    out_ref[:, :, D_EMB:] = jnp.broadcast_to(pe_ref[...][None], (B, BN, DIM))


def kernel(emb, pe):
    pe_rows = jax.lax.slice(pe, (1, 0), (1 + N, DIM))
    grid = (N // BN,)
    return pl.pallas_call(
        _copy_concat_kernel,
        grid=grid,
        in_specs=[
            pl.BlockSpec((B, BN, D_EMB), lambda j: (0, j, 0)),
            pl.BlockSpec((BN, DIM), lambda j: (j, 0)),
        ],
        out_specs=pl.BlockSpec((B, BN, D_EMB + DIM), lambda j: (0, j, 0)),
        out_shape=jax.ShapeDtypeStruct((B, N, D_EMB + DIM), emb.dtype),
    )(emb, pe_rows)
